# Initial kernel scaffold; baseline (speedup 1.0000x reference)
import functools
import jax
import jax.numpy as jnp
from jax import lax
from jax.experimental import pallas as pl
from jax.experimental.pallas import tpu as pltpu
from jax.experimental.pallas import tpu_sc as plsc

B = 4
G = 2048
DIM = 256
BINS = 6
DIM_HEAD = 32
NB_FEAT = 128
E = 40960

NBIN = 342
NLAST = G - 5 * NBIN
NPAD = 352
PTOT = BINS * NPAD
PIDX = 2176

f32 = jnp.float32
i32 = jnp.int32


def _pre_body(x_ref, ln_s, ln_b, wl, wr, y_ref, xl_ref, xr_ref):
    x = x_ref[0]
    mu = jnp.mean(x, axis=-1, keepdims=True)
    var = jnp.mean((x - mu) ** 2, axis=-1, keepdims=True)
    y = (x - mu) / jnp.sqrt(var + 1e-5) * ln_s[...] + ln_b[...]
    y_ref[0] = y
    xl_ref[0] = jnp.dot(y, wl[...], preferred_element_type=f32)
    xr_ref[0] = jnp.dot(y, wr[...], preferred_element_type=f32)


def _pre(x, ln_s, ln_b, wl, wr):
    return pl.pallas_call(
        _pre_body,
        grid=(B,),
        in_specs=[
            pl.BlockSpec((1, G, DIM), lambda b: (b, 0, 0)),
            pl.BlockSpec((DIM,), lambda b: (0,)),
            pl.BlockSpec((DIM,), lambda b: (0,)),
            pl.BlockSpec((DIM, DIM), lambda b: (0, 0)),
            pl.BlockSpec((DIM, DIM), lambda b: (0, 0)),
        ],
        out_specs=[
            pl.BlockSpec((1, G, DIM), lambda b: (b, 0, 0)),
            pl.BlockSpec((1, G, DIM), lambda b: (b, 0, 0)),
            pl.BlockSpec((1, G, DIM), lambda b: (b, 0, 0)),
        ],
        out_shape=[
            jax.ShapeDtypeStruct((B, G, DIM), f32),
            jax.ShapeDtypeStruct((B, G, DIM), f32),
            jax.ShapeDtypeStruct((B, G, DIM), f32),
        ],
    )(x, ln_s, ln_b, wl, wr)


EPW = E // 16
ECH = 128
NCH = EPW // ECH
BPC = B // 2


def _gat_sc(xl_flat, xr_flat, src, dst, att, zrows, zden):
    mesh = plsc.VectorSubcoreMesh(core_axis_name="c", subcore_axis_name="s")

    @functools.partial(
        pl.kernel,
        out_type=[
            jax.ShapeDtypeStruct((B * G, DIM), f32),
            jax.ShapeDtypeStruct((B, G), f32),
        ],
        mesh=mesh,
        scratch_types=[
            pltpu.VMEM((ECH, DIM), f32),
            pltpu.VMEM((ECH, DIM), f32),
            pltpu.VMEM((NCH, ECH), i32),
            pltpu.VMEM((NCH, ECH), i32),
            pltpu.VMEM((NCH, ECH), i32),
            pltpu.VMEM((NCH, ECH), i32),
            pltpu.VMEM((G,), f32),
            pltpu.VMEM((ECH,), f32),
            pltpu.VMEM((DIM,), f32),
            pltpu.VMEM_SHARED((G, DIM), f32),
            pltpu.VMEM_SHARED((G,), f32),
            pltpu.SemaphoreType.DMA,
            pltpu.SemaphoreType.DMA,
        ],
    )
    def k(xl_h, xr_h, src_h, dst_h, att_h, zr_h, zd_h, acc_h, den_h,
          a_buf, b_buf, s_raw, d_raw, s_off, d_off, den_p, e_buf, att_v,
          acc_sh, den_sh, sem0, sem1):
        c = lax.axis_index("c")
        s = lax.axis_index("s")
        pltpu.sync_copy(att_h, att_v)
        ebase = s * EPW
        for j in range(NCH):
            pltpu.sync_copy(src_h.at[pl.ds(ebase + j * ECH, ECH)], s_raw.at[j])
            pltpu.sync_copy(dst_h.at[pl.ds(ebase + j * ECH, ECH)], d_raw.at[j])

        def zero_vec(i, _):
            den_p[pl.ds(i * 16, 16)] = jnp.zeros((16,), f32)
            return 0

        for t in range(BPC):
            b = c * BPC + t
            lax.fori_loop(0, G // 16, zero_vec, 0)
            pltpu.sync_copy(zr_h, acc_sh.at[pl.ds(s * (G // 16), G // 16)])

            @pl.when(s == 0)
            def _():
                pltpu.sync_copy(zd_h, den_sh)

            boff = b * G

            def off_body(i, _):
                j = i // (ECH // 16)
                g = i % (ECH // 16)
                sl = (j, pl.ds(g * 16, 16))
                s_off[sl] = s_raw[sl] + boff
                d_off[sl] = d_raw[sl] + boff
                return 0

            lax.fori_loop(0, NCH * (ECH // 16), off_body, 0)
            plsc.subcore_barrier()

            def chunk_body(j, _):
                cp0 = pltpu.make_async_copy(xl_h.at[s_off.at[j]], a_buf, sem0)
                cp0.start()
                cp1 = pltpu.make_async_copy(xr_h.at[d_off.at[j]], b_buf, sem1)
                cp1.start()
                cp0.wait()
                cp1.wait()

                def edge_body(e, _):
                    acc = jnp.zeros((16,), f32)
                    for g in range(DIM // 16):
                        sl = pl.ds(g * 16, 16)
                        u = a_buf[e, sl] + b_buf[e, sl]
                        m = jnp.where(u >= 0.0, u, 0.2 * u)
                        acc = acc + m * att_v[sl]
                    e_buf[e] = jnp.sum(acc, axis=0)
                    return 0

                lax.fori_loop(0, ECH, edge_body, 0)

                def wgrp(gi, _):
                    sl = pl.ds(gi * 16, 16)
                    wv = jnp.exp(e_buf[sl])
                    e_buf[sl] = wv
                    plsc.addupdate_scatter(den_p, [d_raw[j, sl]], wv)
                    return 0

                lax.fori_loop(0, ECH // 16, wgrp, 0)

                def scale_body(e, _):
                    w = e_buf[e]
                    for g in range(DIM // 16):
                        sl = pl.ds(g * 16, 16)
                        a_buf[e, sl] = a_buf[e, sl] * w
                    return 0

                lax.fori_loop(0, ECH, scale_body, 0)
                pltpu.sync_copy(a_buf, acc_sh.at[d_raw.at[j]], add=True)
                return 0

            lax.fori_loop(0, NCH, chunk_body, 0)
            pltpu.sync_copy(den_p, den_sh, add=True)
            plsc.subcore_barrier()
            rows = G // 16
            pltpu.sync_copy(acc_sh.at[pl.ds(s * rows, rows)],
                            acc_h.at[pl.ds(boff + s * rows, rows)])

            @pl.when(s == 0)
            def _():
                pltpu.sync_copy(den_sh, den_h.at[b])

            plsc.subcore_barrier()

    return k(xl_flat, xr_flat, src, dst, att, zrows, zden)


def _router_body(y_ref, acc_ref, den_ref, bias, wbw, x2_ref, rk_ref, od_ref):
    bidx = pl.program_id(0)
    y = y_ref[0]
    acc = acc_ref[0]
    den = den_ref[...]
    x2 = y + acc / (den.reshape(G, 1) + 1e-16) + bias[...].reshape(1, DIM)
    x2_ref[0] = x2
    s_col = jax.lax.dot_general(x2, wbw[...].reshape(DIM, 1),
                                (((1,), (0,)), ((), ())),
                                preferred_element_type=f32)
    s_row = jax.lax.dot_general(wbw[...].reshape(DIM, 1), x2,
                                (((0,), (1,)), ((), ())),
                                preferred_element_type=f32)
    si = jax.lax.broadcast_in_dim(s_col, (G, G), (0, 1))
    sj = jax.lax.broadcast_in_dim(s_row, (G, G), (0, 1))
    jlt = jax.lax.broadcasted_iota(i32, (G, G), 1) < \
        jax.lax.broadcasted_iota(i32, (G, G), 0)
    beats = jnp.where((sj > si) | ((sj == si) & jlt), 1.0, 0.0)
    rank = jnp.sum(beats, axis=1)
    binid = (jnp.where(rank >= NBIN, 1.0, 0.0)
             + jnp.where(rank >= 2 * NBIN, 1.0, 0.0)
             + jnp.where(rank >= 3 * NBIN, 1.0, 0.0)
             + jnp.where(rank >= 4 * NBIN, 1.0, 0.0)
             + jnp.where(rank >= 5 * NBIN, 1.0, 0.0))
    rp = rank + (NPAD - NBIN) * binid
    rk_ref[...] = (rp + bidx * PTOT).astype(i32).reshape(1, G)
    rp_col = jax.lax.broadcast_in_dim(rp.reshape(G, 1), (G, PIDX), (0, 1))
    p_iota = jax.lax.broadcasted_iota(f32, (G, PIDX), 1)
    onehot = jnp.where(rp_col == p_iota, 1.0, 0.0)
    i_row = jax.lax.broadcasted_iota(f32, (1, G), 1)
    order = jax.lax.dot_general(i_row, onehot, (((1,), (0,)), ((), ())),
                                preferred_element_type=f32)
    od_ref[...] = (order + bidx * G).astype(i32)


def _router(y, acc, den, bias, wbw):
    return pl.pallas_call(
        _router_body,
        grid=(B,),
        in_specs=[
            pl.BlockSpec((1, G, DIM), lambda b: (b, 0, 0)),
            pl.BlockSpec((1, G, DIM), lambda b: (b, 0, 0)),
            pl.BlockSpec((1, G), lambda b: (b, 0)),
            pl.BlockSpec((DIM,), lambda b: (0,)),
            pl.BlockSpec((DIM,), lambda b: (0,)),
        ],
        out_specs=[
            pl.BlockSpec((1, G, DIM), lambda b: (b, 0, 0)),
            pl.BlockSpec((1, G), lambda b: (b, 0)),
            pl.BlockSpec((1, PIDX), lambda b: (b, 0)),
        ],
        out_shape=[
            jax.ShapeDtypeStruct((B, G, DIM), f32),
            jax.ShapeDtypeStruct((B, G), i32),
            jax.ShapeDtypeStruct((B, PIDX), i32),
        ],
    )(y, acc, den, bias, wbw)


def _sc_row_gather(table_flat, idx_flat, n_out, rpw, idx_stride, idx_len):
    mesh = plsc.VectorSubcoreMesh(core_axis_name="c", subcore_axis_name="s")
    wpb = 32 // B

    @functools.partial(
        pl.kernel,
        out_type=jax.ShapeDtypeStruct((n_out, DIM), f32),
        mesh=mesh,
        scratch_types=[
            pltpu.VMEM((rpw,), i32),
            pltpu.VMEM((rpw, DIM), f32),
            pltpu.SemaphoreType.DMA,
        ],
    )
    def k(tab_h, idx_h, out_h, idx_v, rows_v, sem):
        c = lax.axis_index("c")
        s = lax.axis_index("s")
        w = s * 2 + c
        b = w // wpb
        r0 = (w % wpb) * rpw
        pltpu.sync_copy(idx_h.at[pl.ds(b * idx_stride + r0, rpw)], idx_v)
        pltpu.make_async_copy(tab_h.at[idx_v], rows_v, sem).wait()
        pltpu.sync_copy(rows_v, out_h.at[pl.ds(b * idx_len + r0, rpw)])

    return k(table_flat, idx_flat)


def _performer(x, p, valid):
    N = x.shape[0]

    def ln(v, sc, bi):
        mu = jnp.mean(v, axis=-1, keepdims=True)
        var = jnp.mean((v - mu) ** 2, axis=-1, keepdims=True)
        return (v - mu) / jnp.sqrt(var + 1e-5) * sc.reshape(1, -1) + \
            bi.reshape(1, -1)

    y = ln(x, p['ln1_s'], p['ln1_b'])
    q = jnp.dot(y, p['Wq'], preferred_element_type=f32)
    kk = jnp.dot(y, p['Wk'], preferred_element_type=f32)
    v = jnp.dot(y, p['Wv'], preferred_element_type=f32)
    norm = DIM_HEAD ** -0.25
    ratio = NB_FEAT ** -0.5
    if valid is not None:
        rowmask = jax.lax.broadcasted_iota(i32, (N, NB_FEAT), 0) < valid
    outs = []
    ones_col = jnp.ones((N, 1), f32)
    for h in range(2):
        qh = q[:, h * DIM_HEAD:(h + 1) * DIM_HEAD]
        kh = kk[:, h * DIM_HEAD:(h + 1) * DIM_HEAD]
        vh = v[:, h * DIM_HEAD:(h + 1) * DIM_HEAD]
        ddq = jax.lax.dot_general(qh * norm, p['proj'],
                                  (((1,), (1,)), ((), ())),
                                  preferred_element_type=f32)
        diagq = jnp.sum(jnp.square(qh), axis=1, keepdims=True) * \
            (norm ** 2) * 0.5
        qstab = jnp.max(ddq, axis=1, keepdims=True)
        qp = ratio * (jnp.exp(ddq - diagq - qstab) + 1e-4)
        ddk = jax.lax.dot_general(kh * norm, p['proj'],
                                  (((1,), (1,)), ((), ())),
                                  preferred_element_type=f32)
        diagk = jnp.sum(jnp.square(kh), axis=1, keepdims=True) * \
            (norm ** 2) * 0.5
        if valid is not None:
            kstab = jnp.max(jnp.where(rowmask, ddk, -1e30))
        else:
            kstab = jnp.max(ddk)
        kp = ratio * (jnp.exp(ddk - diagk - kstab) + 1e-4)
        if valid is not None:
            kp = jnp.where(rowmask, kp, 0.0)
        ctx = jax.lax.dot_general(kp, vh, (((0,), (0,)), ((), ())),
                                  preferred_element_type=f32)
        ksum = jax.lax.dot_general(kp, ones_col, (((0,), (0,)), ((), ())),
                                   preferred_element_type=f32)
        num = jnp.dot(qp, ctx, preferred_element_type=f32)
        den = jnp.dot(qp, ksum, preferred_element_type=f32)
        outs.append(num / (den + 1e-6))
    a = jnp.concatenate(outs, axis=1)
    a = jnp.dot(a, p['Wo'], preferred_element_type=f32)
    x = x + a
    y2 = ln(x, p['ln2_s'], p['ln2_b'])
    ff = jnp.dot(
        jax.nn.gelu(jnp.dot(y2, p['W1'], preferred_element_type=f32)
                    + p['b1'].reshape(1, -1)),
        p['W2'], preferred_element_type=f32) + p['b2'].reshape(1, -1)
    return x + ff


_PKEYS = ['ln1_s', 'ln1_b', 'Wq', 'Wk', 'Wv', 'Wo', 'proj',
          'ln2_s', 'ln2_b', 'W1', 'b1', 'W2', 'b2']


def _bins_body(x_ref, *refs):
    prefs = refs[:len(_PKEYS)]
    out_ref = refs[len(_PKEYS)]
    p = {k: r[0] for k, r in zip(_PKEYS, prefs)}
    valid = jnp.where(pl.program_id(1) == BINS - 1, NLAST, NBIN)
    out_ref[...] = _performer(x_ref[...], p, valid)


def _bins(xs_pad, pstack):
    in_specs = [pl.BlockSpec((NPAD, DIM), lambda b, i: (b * BINS + i, 0))]
    for kname in _PKEYS:
        shp = pstack[kname].shape
        blk = (1,) + shp[1:]
        nz = len(shp) - 1
        in_specs.append(
            pl.BlockSpec(blk, lambda b, i, _nz=nz: (i,) + (0,) * _nz))
    return pl.pallas_call(
        _bins_body,
        grid=(B, BINS),
        in_specs=in_specs,
        out_specs=pl.BlockSpec((NPAD, DIM), lambda b, i: (b * BINS + i, 0)),
        out_shape=jax.ShapeDtypeStruct((B * PTOT, DIM), f32),
    )(xs_pad, *[pstack[k] for k in _PKEYS])


def _full_body(x_ref, *refs):
    prefs = refs[:len(_PKEYS)]
    out_ref = refs[len(_PKEYS)]
    p = {k: r[...] for k, r in zip(_PKEYS, prefs)}
    out_ref[0] = _performer(x_ref[0], p, None)


def _full(x3, p):
    in_specs = [pl.BlockSpec((1, G, DIM), lambda b: (b, 0, 0))]
    for kname in _PKEYS:
        shp = p[kname].shape
        in_specs.append(
            pl.BlockSpec(shp, lambda b, _nz=len(shp): (0,) * _nz))
    return pl.pallas_call(
        _full_body,
        grid=(B,),
        in_specs=in_specs,
        out_specs=pl.BlockSpec((1, G, DIM), lambda b: (b, 0, 0)),
        out_shape=jax.ShapeDtypeStruct((B, G, DIM), f32),
    )(x3, *[p[k] for k in _PKEYS])


def kernel(x, edge_index, params):
    gp = params['gat']
    y, xl, xr = _pre(x, params['ln_s'], params['ln_b'], gp['Wl'], gp['Wr'])
    src = edge_index[0].astype(i32)
    dst = edge_index[1].astype(i32)
    zrows = jnp.zeros((G // 16, DIM), f32)
    zden = jnp.zeros((G,), f32)
    acc, den = _gat_sc(xl.reshape(B * G, DIM), xr.reshape(B * G, DIM),
                       src, dst, gp['att'], zrows, zden)
    acc = acc.reshape(B, G, DIM)
    x2, rankp, orderp = _router(y, acc, den, gp['bias'],
                                params['wb_W'].reshape(DIM))
    xs_pad = _sc_row_gather(x2.reshape(B * G, DIM), orderp.reshape(-1),
                            B * PTOT, PTOT // 8, PIDX, PTOT)
    pstack = {k: jnp.stack([params['bins'][i][k] for i in range(BINS)])
              for k in _PKEYS}
    xs_out = _bins(xs_pad, pstack)
    x3 = _sc_row_gather(xs_out, rankp.reshape(-1), B * G, G // 8, G, G)
    return _full(x3.reshape(B, G, DIM), params['full'][0])


# speedup vs baseline: 2.5342x; 2.5342x over previous
import functools
import jax
import jax.numpy as jnp
from jax import lax
from jax.experimental import pallas as pl
from jax.experimental.pallas import tpu as pltpu
from jax.experimental.pallas import tpu_sc as plsc

B = 4
G = 2048
DIM = 256
BINS = 6
DIM_HEAD = 32
NB_FEAT = 128
E = 40960

NBIN = 342
NLAST = G - 5 * NBIN
NPAD = 352
PTOT = BINS * NPAD
PIDX = 2176

DIMW = DIM + 16

f32 = jnp.float32
i32 = jnp.int32


def _pre_body(x_ref, ln_s, ln_b, wl, wr, y_ref, xl_ref, xr_ref):
    x = x_ref[0]
    mu = jnp.mean(x, axis=-1, keepdims=True)
    var = jnp.mean((x - mu) ** 2, axis=-1, keepdims=True)
    y = (x - mu) / jnp.sqrt(var + 1e-5) * ln_s[...] + ln_b[...]
    y_ref[0] = y
    xl_ref[0] = jnp.dot(y, wl[...], preferred_element_type=f32)
    xr_ref[0] = jnp.dot(y, wr[...], preferred_element_type=f32)


def _pre(x, ln_s, ln_b, wl, wr):
    return pl.pallas_call(
        _pre_body,
        grid=(B,),
        in_specs=[
            pl.BlockSpec((1, G, DIM), lambda b: (b, 0, 0)),
            pl.BlockSpec((DIM,), lambda b: (0,)),
            pl.BlockSpec((DIM,), lambda b: (0,)),
            pl.BlockSpec((DIM, DIM), lambda b: (0, 0)),
            pl.BlockSpec((DIM, DIM), lambda b: (0, 0)),
        ],
        out_specs=[
            pl.BlockSpec((1, G, DIM), lambda b: (b, 0, 0)),
            pl.BlockSpec((1, G, DIM), lambda b: (b, 0, 0)),
            pl.BlockSpec((1, G, DIM), lambda b: (b, 0, 0)),
        ],
        out_shape=[
            jax.ShapeDtypeStruct((B, G, DIM), f32),
            jax.ShapeDtypeStruct((B, G, DIM), f32),
            jax.ShapeDtypeStruct((B, G, DIM), f32),
        ],
    )(x, ln_s, ln_b, wl, wr)


ECH = 128
BPC = B // 2
GPT = G // 16


def _gat_sc(xl_flat, xr_flat, src_s, dloc, bounds, att):
    mesh = plsc.VectorSubcoreMesh(core_axis_name="c", subcore_axis_name="s")

    @functools.partial(
        pl.kernel,
        out_type=jax.ShapeDtypeStruct((B * G, DIMW), f32),
        mesh=mesh,
        scratch_types=[
            pltpu.VMEM((ECH, DIM), f32),
            pltpu.VMEM((ECH, DIM), f32),
            pltpu.VMEM((GPT, DIMW), f32),
            pltpu.VMEM((ECH,), i32),
            pltpu.VMEM((ECH,), i32),
            pltpu.VMEM((ECH,), i32),
            pltpu.VMEM((DIM,), f32),
            pltpu.VMEM((16,), i32),
            pltpu.SemaphoreType.DMA,
            pltpu.SemaphoreType.DMA,
        ],
    )
    def k(xl_h, xr_h, src_h, dloc_h, bnd_h, att_h, acc_h,
          a_buf, b_buf, acc_v, sidx, d_buf, didx, att_v, bnd, sem0, sem1):
        c = lax.axis_index("c")
        s = lax.axis_index("s")
        pltpu.sync_copy(att_h, att_v)
        pltpu.sync_copy(bnd_h.at[s], bnd)
        bv = bnd[...]
        lo = bv[0]
        hi = bv[1]
        klo = lo // ECH
        khi = (hi + (ECH - 1)) // ECH
        lanes = lax.iota(i32, 16)

        for t in range(BPC):
            b = c * BPC + t
            boff = b * G

            def zacc(i, _):
                r = i // (DIMW // 16)
                g = i % (DIMW // 16)
                acc_v[r, pl.ds(g * 16, 16)] = jnp.zeros((16,), f32)
                return 0

            lax.fori_loop(0, GPT * (DIMW // 16), zacc, 0)

            def chunk_body(kc, _):
                ebase = kc * ECH
                pltpu.sync_copy(src_h.at[pl.ds(ebase, ECH)], sidx)
                pltpu.sync_copy(dloc_h.at[pl.ds(ebase, ECH)], d_buf)

                def off_body(g16, _):
                    sl = pl.ds(g16 * 16, 16)
                    sidx[sl] = sidx[sl] + boff
                    didx[sl] = d_buf[sl] + (boff + s * GPT)
                    return 0

                lax.fori_loop(0, ECH // 16, off_body, 0)
                cp0 = pltpu.make_async_copy(xl_h.at[sidx], a_buf, sem0)
                cp0.start()
                cp1 = pltpu.make_async_copy(xr_h.at[didx], b_buf, sem1)
                cp1.start()
                cp0.wait()
                cp1.wait()

                def grp_body(g16, _):
                    dv = d_buf[pl.ds(g16 * 16, 16)]
                    for l in range(16):
                        e = g16 * 16 + l
                        accr = jnp.zeros((16,), f32)
                        for g in range(DIM // 16):
                            sl = pl.ds(g * 16, 16)
                            u = a_buf[e, sl] + b_buf[e, sl]
                            m = jnp.where(u >= 0.0, u, 0.2 * u)
                            accr = accr + m * att_v[sl]
                        esum = accr[0]
                        for ll in range(1, 16):
                            esum = esum + accr[ll]
                        ge = ebase + e
                        valid = jnp.where((ge >= lo) & (ge < hi), 1.0, 0.0)
                        wm = jnp.exp(jax.lax.broadcast(esum, (16,))) * \
                            jax.lax.broadcast(valid, (16,))
                        dl = dv[l]
                        for g in range(DIM // 16):
                            sl = pl.ds(g * 16, 16)
                            acc_v[dl, sl] = acc_v[dl, sl] + \
                                a_buf[e, sl] * wm
                        tl = pl.ds(DIM, 16)
                        acc_v[dl, tl] = acc_v[dl, tl] + jnp.where(
                            lanes == 0, wm, 0.0)
                    return 0

                lax.fori_loop(0, ECH // 16, grp_body, 0)
                return 0

            lax.fori_loop(klo, khi, chunk_body, 0)
            pltpu.sync_copy(acc_v, acc_h.at[pl.ds(boff + s * GPT, GPT)])

    return k(xl_flat, xr_flat, src_s, dloc, bounds, att)


def _edge_prep(src, dst):
    order_e = jnp.argsort(dst)
    src_s = src[order_e]
    dst_s = dst[order_e]
    dloc = jnp.mod(dst_s, GPT).astype(i32)
    estart = jnp.searchsorted(dst_s, jnp.arange(17, dtype=i32) * GPT,
                              side='left').astype(i32)
    bounds = jnp.zeros((16, 16), i32)
    bounds = bounds.at[:, 0].set(estart[:16]).at[:, 1].set(estart[1:])
    return src_s.astype(i32), dloc, bounds


def _scores_body(y_ref, acc_ref, bias, wbw, x2_ref, s_ref):
    y = y_ref[0]
    accw = acc_ref[0]
    acc = accw[:, :DIM]
    tail = accw[:, DIM:DIM + 16]
    tmask = jax.lax.broadcasted_iota(i32, (G, 16), 1) == 0
    den = jnp.sum(jnp.where(tmask, tail, 0.0), axis=1, keepdims=True)
    x2 = y + acc / (den + 1e-16) + bias[...].reshape(1, DIM)
    x2_ref[0] = x2
    s_ref[0] = jax.lax.dot_general(x2, wbw[...].reshape(DIM, 1),
                                   (((1,), (0,)), ((), ())),
                                   preferred_element_type=f32)


def _scores(y, accw, bias, wbw):
    return pl.pallas_call(
        _scores_body,
        grid=(B,),
        in_specs=[
            pl.BlockSpec((1, G, DIM), lambda b: (b, 0, 0)),
            pl.BlockSpec((1, G, DIMW), lambda b: (b, 0, 0)),
            pl.BlockSpec((DIM,), lambda b: (0,)),
            pl.BlockSpec((DIM,), lambda b: (0,)),
        ],
        out_specs=[
            pl.BlockSpec((1, G, DIM), lambda b: (b, 0, 0)),
            pl.BlockSpec((1, G, 1), lambda b: (b, 0, 0)),
        ],
        out_shape=[
            jax.ShapeDtypeStruct((B, G, DIM), f32),
            jax.ShapeDtypeStruct((B, G, 1), f32),
        ],
    )(y, accw, bias, wbw)


def _rank_body(sc_ref, sr_ref, rk_ref, od_ref):
    bidx = pl.program_id(0)
    s_col = sc_ref[0]
    s_row = sr_ref[0]
    si = jax.lax.broadcast_in_dim(s_col, (G, G), (0, 1))
    sj = jax.lax.broadcast_in_dim(s_row, (G, G), (0, 1))
    jlt = jax.lax.broadcasted_iota(i32, (G, G), 1) < \
        jax.lax.broadcasted_iota(i32, (G, G), 0)
    beats = jnp.where((sj > si) | ((sj == si) & jlt), 1.0, 0.0)
    rank = jnp.sum(beats, axis=1, keepdims=True)
    binid = (jnp.where(rank >= NBIN, 1.0, 0.0)
             + jnp.where(rank >= 2 * NBIN, 1.0, 0.0)
             + jnp.where(rank >= 3 * NBIN, 1.0, 0.0)
             + jnp.where(rank >= 4 * NBIN, 1.0, 0.0)
             + jnp.where(rank >= 5 * NBIN, 1.0, 0.0))
    rp = rank + (NPAD - NBIN) * binid
    rk_ref[...] = (rp + (bidx * PTOT + 0.5)).astype(i32).reshape(1, G, 1)
    rp_col = jax.lax.broadcast_in_dim(rp, (G, PIDX), (0, 1))
    p_iota = jax.lax.broadcasted_iota(i32, (G, PIDX), 1).astype(f32)
    onehot = jnp.where(rp_col == p_iota, 1.0, 0.0)
    i_row = jax.lax.broadcasted_iota(i32, (1, G), 1).astype(f32)
    order = jax.lax.dot_general(i_row, onehot, (((1,), (0,)), ((), ())),
                                preferred_element_type=f32)
    od_ref[...] = (order + (bidx * G + 0.5)).astype(i32).reshape(1, 1, PIDX)


def _rank(scores):
    return pl.pallas_call(
        _rank_body,
        grid=(B,),
        in_specs=[
            pl.BlockSpec((1, G, 1), lambda b: (b, 0, 0)),
            pl.BlockSpec((1, 1, G), lambda b: (b, 0, 0)),
        ],
        out_specs=[
            pl.BlockSpec((1, G, 1), lambda b: (b, 0, 0)),
            pl.BlockSpec((1, 1, PIDX), lambda b: (b, 0, 0)),
        ],
        out_shape=[
            jax.ShapeDtypeStruct((B, G, 1), i32),
            jax.ShapeDtypeStruct((B, 1, PIDX), i32),
        ],
    )(scores, scores.reshape(B, 1, G))


def _rank_xla(scores):
    sc = scores.reshape(B, G)
    rank = jnp.sum(
        (sc[:, None, :] > sc[:, :, None])
        | ((sc[:, None, :] == sc[:, :, None])
           & (jnp.arange(G)[None, None, :] < jnp.arange(G)[None, :, None])),
        axis=2).astype(i32)
    binid = jnp.clip(rank // NBIN, 0, 5)
    rp = rank + (NPAD - NBIN) * binid
    rk = rp + jnp.arange(B)[:, None] * PTOT
    od = jnp.zeros((B, PIDX), i32).at[
        jnp.arange(B)[:, None], rp].set(jnp.arange(G)[None, :])
    od = od + jnp.arange(B)[:, None] * G
    return rk.reshape(B, G, 1), od.reshape(B, 1, PIDX)


def _router(y, accw, bias, wbw):
    x2, scores = _scores(y, accw, bias, wbw)
    rankp, orderp = _rank_xla(scores)
    return x2, rankp, orderp


def _sc_row_gather(table_flat, idx_flat, n_out, cs, ncpw, idx_stride,
                   idx_len):
    mesh = plsc.VectorSubcoreMesh(core_axis_name="c", subcore_axis_name="s")
    wpb = 32 // B
    rpw = cs * ncpw

    @functools.partial(
        pl.kernel,
        out_type=jax.ShapeDtypeStruct((n_out, DIM), f32),
        mesh=mesh,
        scratch_types=[
            pltpu.VMEM((rpw,), i32),
            pltpu.VMEM((cs, DIM), f32),
            pltpu.SemaphoreType.DMA,
        ],
    )
    def k(tab_h, idx_h, out_h, idx_v, rows_v, sem):
        c = lax.axis_index("c")
        s = lax.axis_index("s")
        w = s * 2 + c
        b = w // wpb
        r0 = (w % wpb) * rpw
        pltpu.sync_copy(idx_h.at[pl.ds(b * idx_stride + r0, rpw)], idx_v)

        def chunk(kk, _):
            cp = pltpu.make_async_copy(
                tab_h.at[idx_v.at[pl.ds(kk * cs, cs)]], rows_v, sem)
            cp.start()
            cp.wait()
            pltpu.sync_copy(
                rows_v, out_h.at[pl.ds(b * idx_len + r0 + kk * cs, cs)])
            return 0

        lax.fori_loop(0, ncpw, chunk, 0)

    return k(table_flat, idx_flat)


def _performer(x, p, valid):
    N = x.shape[0]

    def ln(v, sc, bi):
        mu = jnp.mean(v, axis=-1, keepdims=True)
        var = jnp.mean((v - mu) ** 2, axis=-1, keepdims=True)
        return (v - mu) / jnp.sqrt(var + 1e-5) * sc.reshape(1, -1) + \
            bi.reshape(1, -1)

    y = ln(x, p['ln1_s'], p['ln1_b'])
    q = jnp.dot(y, p['Wq'], preferred_element_type=f32)
    kk = jnp.dot(y, p['Wk'], preferred_element_type=f32)
    v = jnp.dot(y, p['Wv'], preferred_element_type=f32)
    norm = DIM_HEAD ** -0.25
    ratio = NB_FEAT ** -0.5
    if valid is not None:
        rowmask = jax.lax.broadcasted_iota(i32, (N, NB_FEAT), 0) < valid
    outs = []
    ones_col = jnp.ones((N, 1), f32)
    for h in range(2):
        qh = q[:, h * DIM_HEAD:(h + 1) * DIM_HEAD]
        kh = kk[:, h * DIM_HEAD:(h + 1) * DIM_HEAD]
        vh = v[:, h * DIM_HEAD:(h + 1) * DIM_HEAD]
        ddq = jax.lax.dot_general(qh * norm, p['proj'],
                                  (((1,), (1,)), ((), ())),
                                  preferred_element_type=f32)
        diagq = jnp.sum(jnp.square(qh), axis=1, keepdims=True) * \
            (norm ** 2) * 0.5
        qstab = jnp.max(ddq, axis=1, keepdims=True)
        qp = ratio * (jnp.exp(ddq - diagq - qstab) + 1e-4)
        ddk = jax.lax.dot_general(kh * norm, p['proj'],
                                  (((1,), (1,)), ((), ())),
                                  preferred_element_type=f32)
        diagk = jnp.sum(jnp.square(kh), axis=1, keepdims=True) * \
            (norm ** 2) * 0.5
        if valid is not None:
            kstab = jnp.max(jnp.where(rowmask, ddk, -1e30))
        else:
            kstab = jnp.max(ddk)
        kp = ratio * (jnp.exp(ddk - diagk - kstab) + 1e-4)
        if valid is not None:
            kp = jnp.where(rowmask, kp, 0.0)
        ctx = jax.lax.dot_general(kp, vh, (((0,), (0,)), ((), ())),
                                  preferred_element_type=f32)
        ksum = jax.lax.dot_general(kp, ones_col, (((0,), (0,)), ((), ())),
                                   preferred_element_type=f32)
        num = jnp.dot(qp, ctx, preferred_element_type=f32)
        den = jnp.dot(qp, ksum, preferred_element_type=f32)
        outs.append(num / (den + 1e-6))
    a = jnp.concatenate(outs, axis=1)
    a = jnp.dot(a, p['Wo'], preferred_element_type=f32)
    x = x + a
    y2 = ln(x, p['ln2_s'], p['ln2_b'])
    ff = jnp.dot(
        jax.nn.gelu(jnp.dot(y2, p['W1'], preferred_element_type=f32)
                    + p['b1'].reshape(1, -1)),
        p['W2'], preferred_element_type=f32) + p['b2'].reshape(1, -1)
    return x + ff


_PKEYS = ['ln1_s', 'ln1_b', 'Wq', 'Wk', 'Wv', 'Wo', 'proj',
          'ln2_s', 'ln2_b', 'W1', 'b1', 'W2', 'b2']


def _bins_body(x_ref, *refs):
    prefs = refs[:len(_PKEYS)]
    out_ref = refs[len(_PKEYS)]
    p = {k: r[0] for k, r in zip(_PKEYS, prefs)}
    valid = jnp.where(pl.program_id(1) == BINS - 1, NLAST, NBIN)
    out_ref[...] = _performer(x_ref[...], p, valid)


def _bins(xs_pad, pstack):
    in_specs = [pl.BlockSpec((NPAD, DIM), lambda b, i: (b * BINS + i, 0))]
    for kname in _PKEYS:
        shp = pstack[kname].shape
        blk = (1,) + shp[1:]
        nz = len(shp) - 1
        in_specs.append(
            pl.BlockSpec(blk, lambda b, i, _nz=nz: (i,) + (0,) * _nz))
    return pl.pallas_call(
        _bins_body,
        grid=(B, BINS),
        in_specs=in_specs,
        out_specs=pl.BlockSpec((NPAD, DIM), lambda b, i: (b * BINS + i, 0)),
        out_shape=jax.ShapeDtypeStruct((B * PTOT, DIM), f32),
    )(xs_pad, *[pstack[k] for k in _PKEYS])


def _full_body(x_ref, *refs):
    prefs = refs[:len(_PKEYS)]
    out_ref = refs[len(_PKEYS)]
    p = {k: r[...] for k, r in zip(_PKEYS, prefs)}
    out_ref[0] = _performer(x_ref[0], p, None)


def _full(x3, p):
    in_specs = [pl.BlockSpec((1, G, DIM), lambda b: (b, 0, 0))]
    for kname in _PKEYS:
        shp = p[kname].shape
        in_specs.append(
            pl.BlockSpec(shp, lambda b, _nz=len(shp): (0,) * _nz))
    return pl.pallas_call(
        _full_body,
        grid=(B,),
        in_specs=in_specs,
        out_specs=pl.BlockSpec((1, G, DIM), lambda b: (b, 0, 0)),
        out_shape=jax.ShapeDtypeStruct((B, G, DIM), f32),
    )(x3, *[p[k] for k in _PKEYS])


def kernel(x, edge_index, params):
    gp = params['gat']
    y, xl, xr = _pre(x, params['ln_s'], params['ln_b'], gp['Wl'], gp['Wr'])
    src = edge_index[0].astype(i32)
    dst = edge_index[1].astype(i32)
    src_s, dloc, bounds = _edge_prep(src, dst)
    accw = _gat_sc(xl.reshape(B * G, DIM), xr.reshape(B * G, DIM),
                   src_s, dloc, bounds, gp['att'])
    x2, rankp, orderp = _router(y, accw.reshape(B, G, DIMW), gp['bias'],
                                params['wb_W'].reshape(DIM))
    xs_pad = _sc_row_gather(x2.reshape(B * G, DIM), orderp.reshape(-1),
                            B * PTOT, 88, 3, PIDX, PTOT)
    pstack = {}
    for k in _PKEYS:
        s = jnp.stack([params['bins'][i][k] for i in range(BINS)])
        if s.ndim == 2:
            s = s.reshape(BINS, 1, -1)
        pstack[k] = s
    xs_out = _bins(xs_pad, pstack)
    x3 = _sc_row_gather(xs_out, rankp.reshape(-1), B * G, 128, 2, G, G)
    return _full(x3.reshape(B, G, DIM), params['full'][0])
